# baseline (speedup 1.0000x reference)
import jax
import jax.numpy as jnp
from jax import lax
from jax.experimental import pallas as pl
from jax.experimental.pallas import tpu as pltpu
from jax.experimental.pallas import tpu_sc as plsc

B, N, D = 4, 2048, 1024
F = 2048
E = 8
T = B * N
TM = 256
NET = 2 * T // TM + E
GROWS = NET * TM
TMR = 1024
TMS = 1024
NW = 32
TOKW = T // NW
NTM = 80
NEG = -1e30


def _router_body(x_ref, wr_ref, br_ref, e01_ref, w01_ref, rank_ref, cnt_ref,
                 aux_ref, acc_ref):
    i = pl.program_id(0)
    nb = pl.num_programs(0)
    x = x_ref[...]
    logits = jnp.dot(
        x.astype(jnp.bfloat16), wr_ref[...].astype(jnp.bfloat16),
        preferred_element_type=jnp.float32) + br_ref[...]
    mx = jnp.max(logits, axis=-1, keepdims=True)
    ex = jnp.exp(logits - mx)
    z = jnp.sum(ex, axis=-1, keepdims=True)
    probs = ex / z
    lane = jax.lax.broadcasted_iota(jnp.int32, logits.shape, 1)
    i0 = jnp.min(jnp.where(logits == mx, lane, E), axis=-1, keepdims=True)
    logits1 = jnp.where(lane == i0, NEG, logits)
    m1 = jnp.max(logits1, axis=-1, keepdims=True)
    i1 = jnp.min(jnp.where(logits1 == m1, lane, E), axis=-1, keepdims=True)
    p0 = 1.0 / z
    p1 = jnp.exp(m1 - mx) / z
    denom = p0 + p1 + 1e-9
    w0 = p0 / denom
    w1 = p1 / denom

    @pl.when(i == 0)
    def _():
        acc_ref[...] = jnp.zeros_like(acc_ref)

    oh0 = lane == i0
    oh1 = lane == i1
    oh01 = (oh0 | oh1).astype(jnp.bfloat16)
    r_iota = jax.lax.broadcasted_iota(jnp.int32, (TMR, TMR), 0)
    c_iota = jax.lax.broadcasted_iota(jnp.int32, (TMR, TMR), 1)
    ltri = (c_iota < r_iota).astype(jnp.bfloat16)
    cexcl = jnp.dot(ltri, oh01, preferred_element_type=jnp.float32)
    rankmat = cexcl + acc_ref[1:2, 0:E]
    rank0 = jnp.sum(jnp.where(oh0, rankmat, 0.0), axis=-1, keepdims=True)
    rank1 = jnp.sum(jnp.where(oh1, rankmat, 0.0), axis=-1, keepdims=True)

    lane2 = jax.lax.broadcasted_iota(jnp.int32, (TMR, 2), 1)
    e01_ref[...] = jnp.where(lane2 == 0, i0, i1)
    w01_ref[...] = jnp.where(lane2 == 0, w0, w1)
    rank_ref[...] = jnp.where(lane2 == 0, rank0, rank1).astype(jnp.int32)

    lane16 = jax.lax.broadcasted_iota(jnp.int32, (TMR, 16), 1)
    cnt16 = jnp.sum(
        (lane16 == i0).astype(jnp.float32) + (lane16 == i1).astype(jnp.float32),
        axis=0, keepdims=True)
    ps = jnp.sum(probs, axis=0, keepdims=True)
    acc_ref[0:1, 0:E] += ps
    acc_ref[1:2, 0:16] += cnt16

    @pl.when(i == nb - 1)
    def _():
        cnt_ref[...] = acc_ref[1:2, 0:16].astype(jnp.int32)
        imp = acc_ref[0:1, 0:E] / T
        load = acc_ref[1:2, 0:E] / (T * 2)
        aux_ref[...] = E * jnp.sum(imp * load, axis=-1, keepdims=True)


def _router(x_flat, Wr, br2):
    return pl.pallas_call(
        _router_body,
        grid=(T // TMR,),
        in_specs=[
            pl.BlockSpec((TMR, D), lambda i: (i, 0)),
            pl.BlockSpec((D, E), lambda i: (0, 0)),
            pl.BlockSpec((1, E), lambda i: (0, 0)),
        ],
        out_specs=[
            pl.BlockSpec((TMR, 2), lambda i: (i, 0)),
            pl.BlockSpec((TMR, 2), lambda i: (i, 0)),
            pl.BlockSpec((TMR, 2), lambda i: (i, 0)),
            pl.BlockSpec((1, 16), lambda i: (0, 0)),
            pl.BlockSpec((1, 1), lambda i: (0, 0)),
        ],
        out_shape=[
            jax.ShapeDtypeStruct((T, 2), jnp.int32),
            jax.ShapeDtypeStruct((T, 2), jnp.float32),
            jax.ShapeDtypeStruct((T, 2), jnp.int32),
            jax.ShapeDtypeStruct((1, 16), jnp.int32),
            jax.ShapeDtypeStruct((1, 1), jnp.float32),
        ],
        scratch_shapes=[pltpu.VMEM((8, 128), jnp.float32)],
    )(x_flat, Wr, br2)


def _dispatch_body(x_hbm, e01_hbm, rank_hbm, w01_hbm, cnt_hbm,
                   gx_hbm, sw_hbm, tmap_hbm, dstw_hbm,
                   cnt_v, pstart_v, e_v, rank_v, w_v, dst_v, xb0, xb1,
                   tmap_v, semr, sems, semw):
    wid = lax.axis_index("s") * 2 + lax.axis_index("c")
    tok0 = wid * TOKW
    pltpu.sync_copy(cnt_hbm, cnt_v)
    cnt = cnt_v[...]
    rup = jnp.bitwise_and(cnt + (TM - 1), -TM)
    lane0 = lax.broadcasted_iota(jnp.int32, (16,), 0)
    pstart = jnp.zeros((16,), jnp.int32)
    for e in range(E):
        pe = jnp.sum(jnp.where(lane0 < e, rup, 0))
        pstart = pstart + jnp.where(lane0 == e, pe, 0)
    pstart_v[...] = pstart
    for k in range(2):
        pltpu.sync_copy(e01_hbm.at[k, pl.ds(tok0, TOKW)],
                        e_v.at[pl.ds(k * TOKW, TOKW)])
        pltpu.sync_copy(rank_hbm.at[k, pl.ds(tok0, TOKW)],
                        rank_v.at[pl.ds(k * TOKW, TOKW)])
        pltpu.sync_copy(w01_hbm.at[k, pl.ds(tok0, TOKW)],
                        w_v.at[pl.ds(k * TOKW, TOKW)])

    for r in range(16):
        for v in range(2):
            base = r * 32 + v * 16
            ev = e_v[pl.ds(base, 16)]
            rv = rank_v[pl.ds(base, 16)]
            ps = plsc.load_gather(pstart_v, [ev])
            dst_v[r, pl.ds(v * 16, 16)] = ps + rv
    pltpu.sync_copy(dst_v, dstw_hbm.at[wid])

    wh = [pltpu.async_copy(w_v.at[pl.ds(r * 32, 32)],
                           sw_hbm.at[dst_v.at[r]], semw) for r in range(16)]
    xb = (xb0, xb1)
    rh = [None] * 8
    sh = [None] * 8
    rh[0] = pltpu.async_copy(x_hbm.at[pl.ds(tok0, 32)], xb[0], semr)
    for c in range(8):
        rh[c].wait()
        sh[c] = (pltpu.async_copy(xb[c % 2], gx_hbm.at[dst_v.at[c]], sems),
                 pltpu.async_copy(xb[c % 2], gx_hbm.at[dst_v.at[8 + c]], sems))
        if c + 1 < 8:
            if c >= 1:
                sh[c - 1][0].wait()
                sh[c - 1][1].wait()
            t1 = tok0 + (c + 1) * 32
            rh[c + 1] = pltpu.async_copy(x_hbm.at[pl.ds(t1, 32)],
                                         xb[(c + 1) % 2], semr)
    sh[6][0].wait()
    sh[6][1].wait()
    sh[7][0].wait()
    sh[7][1].wait()
    for h in wh:
        h.wait()

    @pl.when(wid == 0)
    def _():
        tiles = jnp.right_shift(rup, 8)
        lane = lax.broadcasted_iota(jnp.int32, (16,), 0)
        for v in range(NTM // 16):
            acc = jnp.zeros((16,), jnp.int32)
            iv = lane + v * 16
            for e in range(E):
                ce = jnp.sum(jnp.where(lane <= e, tiles, 0))
                acc = acc + (iv >= ce).astype(jnp.int32)
            tmap_v[pl.ds(v * 16, 16)] = acc
        pltpu.sync_copy(tmap_v, tmap_hbm)


def _dispatch(x_flat, e01, rank, w01, cnt16):
    mesh = plsc.VectorSubcoreMesh(core_axis_name="c", subcore_axis_name="s")
    f = pl.kernel(
        _dispatch_body,
        mesh=mesh,
        compiler_params=pltpu.CompilerParams(needs_layout_passes=False),
        out_type=[
            jax.ShapeDtypeStruct((GROWS, D // 2), jnp.int32),
            jax.ShapeDtypeStruct((GROWS,), jnp.float32),
            jax.ShapeDtypeStruct((NTM,), jnp.int32),
            jax.ShapeDtypeStruct((NW, 16, 32), jnp.int32),
        ],
        scratch_types=[
            pltpu.VMEM((16,), jnp.int32),
            pltpu.VMEM((16,), jnp.int32),
            pltpu.VMEM((2 * TOKW,), jnp.int32),
            pltpu.VMEM((2 * TOKW,), jnp.int32),
            pltpu.VMEM((2 * TOKW,), jnp.float32),
            pltpu.VMEM((16, 32), jnp.int32),
            pltpu.VMEM((32, D // 2), jnp.int32),
            pltpu.VMEM((32, D // 2), jnp.int32),
            pltpu.VMEM((NTM,), jnp.int32),
            pltpu.SemaphoreType.DMA,
            pltpu.SemaphoreType.DMA,
            pltpu.SemaphoreType.DMA,
        ],
    )
    return f(x_flat, e01, rank, w01, cnt16)


def _ffn_body(tmap_ref, gx_ref, sw_ref, w1_ref, b1_ref, w2_ref, b2_ref,
              out_ref):
    x = gx_ref[...]
    h = jnp.maximum(
        jnp.dot(x, w1_ref[0], preferred_element_type=jnp.float32) + b1_ref[0],
        0.0)
    o = jnp.dot(h.astype(jnp.bfloat16), w2_ref[0],
                preferred_element_type=jnp.float32) + b2_ref[0]
    out_ref[...] = (o * sw_ref[...]).astype(jnp.bfloat16)


def _ffn(tmap, gx, sw2, W1all, b1all, W2all, b2all):
    grid_spec = pltpu.PrefetchScalarGridSpec(
        num_scalar_prefetch=1,
        grid=(NET,),
        in_specs=[
            pl.BlockSpec((TM, D), lambda i, tm: (i, 0)),
            pl.BlockSpec((TM, 1), lambda i, tm: (i, 0)),
            pl.BlockSpec((1, D, F), lambda i, tm: (tm[i], 0, 0)),
            pl.BlockSpec((1, 1, F), lambda i, tm: (tm[i], 0, 0)),
            pl.BlockSpec((1, F, D), lambda i, tm: (tm[i], 0, 0)),
            pl.BlockSpec((1, 1, D), lambda i, tm: (tm[i], 0, 0)),
        ],
        out_specs=pl.BlockSpec((TM, D), lambda i, tm: (i, 0)),
    )
    return pl.pallas_call(
        _ffn_body,
        grid_spec=grid_spec,
        out_shape=jax.ShapeDtypeStruct((GROWS, D), jnp.bfloat16),
    )(tmap, gx, sw2, W1all, b1all, W2all, b2all)


def _sffn_body(x_ref, ws1_ref, bs1_ref, ws2_ref, bs2_ref, out_ref):
    x = x_ref[...].astype(jnp.bfloat16)
    h = jnp.maximum(
        jnp.dot(x, ws1_ref[...], preferred_element_type=jnp.float32)
        + bs1_ref[...], 0.0)
    out_ref[...] = (jnp.dot(
        h.astype(jnp.bfloat16), ws2_ref[...],
        preferred_element_type=jnp.float32) + bs2_ref[...]).astype(
            jnp.bfloat16)


def _sffn(x_flat, Ws1b, bs1r, Ws2b, bs2r):
    return pl.pallas_call(
        _sffn_body,
        grid=(T // TMS,),
        in_specs=[
            pl.BlockSpec((TMS, D), lambda i: (i, 0)),
            pl.BlockSpec((D, F), lambda i: (0, 0)),
            pl.BlockSpec((1, F), lambda i: (0, 0)),
            pl.BlockSpec((F, D), lambda i: (0, 0)),
            pl.BlockSpec((1, D), lambda i: (0, 0)),
        ],
        out_specs=pl.BlockSpec((TMS, D), lambda i: (i, 0)),
        out_shape=jax.ShapeDtypeStruct((T, D), jnp.bfloat16),
    )(x_flat, Ws1b, bs1r, Ws2b, bs2r)


def _combine_body(fo_hbm, sh_hbm, dstw_hbm, y_hbm,
                  dst_v, b0a, b0b, b1a, b1b, bs_, ya, yb_,
                  semg, semy):
    wid = lax.axis_index("s") * 2 + lax.axis_index("c")
    tok0 = wid * TOKW
    pltpu.sync_copy(dstw_hbm.at[wid], dst_v)
    b0 = (b0a, b0b)
    b1 = (b1a, b1b)
    yb = (ya, yb_)

    def _issue(c):
        cc, hh = c // 2, c % 2
        p = c % 2
        return (
            pltpu.async_copy(fo_hbm.at[dst_v.at[cc, pl.ds(hh * 16, 16)]],
                             b0[p], semg),
            pltpu.async_copy(fo_hbm.at[dst_v.at[8 + cc, pl.ds(hh * 16, 16)]],
                             b1[p], semg),
        )

    gh = [None] * 16
    wrh = [None] * 16
    gh[0] = _issue(0)
    for c in range(16):
        for h in gh[c]:
            h.wait()
        if c + 1 < 16:
            gh[c + 1] = _issue(c + 1)
        p = c % 2
        pltpu.sync_copy(sh_hbm.at[pl.ds(tok0 + c * 16, 16)], bs_)
        if c >= 2:
            wrh[c - 2].wait()

        def _row(i, _, p=p):
            def _col(j, _, i=i):
                sl = pl.ds(j * 16, 16)
                a = plsc.bitcast(b0[p][i, sl], jnp.bfloat16)
                b = plsc.bitcast(b1[p][i, sl], jnp.bfloat16)
                cshared = plsc.bitcast(bs_[i, sl], jnp.bfloat16)
                yb[p][i, sl] = plsc.bitcast(a + b + cshared, jnp.int32)
                return 0
            return lax.fori_loop(0, D // 32, _col, 0)

        lax.fori_loop(0, 16, _row, 0)
        wrh[c] = pltpu.async_copy(yb[p], y_hbm.at[pl.ds(tok0 + c * 16, 16)],
                                  semy)
    wrh[14].wait()
    wrh[15].wait()


def _combine(fo, sh, dstw):
    mesh = plsc.VectorSubcoreMesh(core_axis_name="c", subcore_axis_name="s")
    f = pl.kernel(
        _combine_body,
        mesh=mesh,
        compiler_params=pltpu.CompilerParams(needs_layout_passes=False),
        out_type=jax.ShapeDtypeStruct((T, D // 2), jnp.int32),
        scratch_types=[
            pltpu.VMEM((16, 32), jnp.int32),
            pltpu.VMEM((16, D // 2), jnp.int32),
            pltpu.VMEM((16, D // 2), jnp.int32),
            pltpu.VMEM((16, D // 2), jnp.int32),
            pltpu.VMEM((16, D // 2), jnp.int32),
            pltpu.VMEM((16, D // 2), jnp.int32),
            pltpu.VMEM((16, D // 2), jnp.int32),
            pltpu.VMEM((16, D // 2), jnp.int32),
            pltpu.SemaphoreType.DMA,
            pltpu.SemaphoreType.DMA,
        ],
    )
    return f(fo, sh, dstw)


def kernel(x, Wr, br, W1, b1, W2, b2, Ws1, bs1, Ws2, bs2):
    x_flat = x.reshape(T, D)
    W1all = jnp.concatenate([W1, Ws1[None]], axis=0).astype(jnp.bfloat16)
    W2all = jnp.concatenate([W2, Ws2[None]], axis=0).astype(jnp.bfloat16)
    b1all = jnp.concatenate([b1, bs1[None]], axis=0).reshape(E + 1, 1, F)
    b2all = jnp.concatenate([b2, bs2[None]], axis=0).reshape(E + 1, 1, D)
    br2 = br.reshape(1, E)

    e01t, w01t, rankt, cnt16, aux = _router(x_flat, Wr, br2)
    e01 = e01t.T
    w01 = w01t.T
    rank = rankt.T
    xi = lax.bitcast_convert_type(
        x_flat.astype(jnp.bfloat16).reshape(T, D // 2, 2), jnp.int32)
    gxi, sw, tmap, dstw = _dispatch(xi, e01, rank, w01, cnt16.reshape(16))
    gx = lax.bitcast_convert_type(gxi, jnp.bfloat16).reshape(GROWS, D)
    sh = _sffn(x_flat, Ws1.astype(jnp.bfloat16), bs1.reshape(1, F),
               Ws2.astype(jnp.bfloat16), bs2.reshape(1, D))
    fo = _ffn(tmap, gx, sw.reshape(GROWS, 1), W1all, b1all, W2all, b2all)
    foi = lax.bitcast_convert_type(
        fo.reshape(GROWS, D // 2, 2), jnp.int32)
    shi = lax.bitcast_convert_type(sh.reshape(T, D // 2, 2), jnp.int32)
    yi = _combine(foi, shi, dstw)
    y = lax.bitcast_convert_type(yi, jnp.bfloat16).reshape(T, D)
    return y.astype(jnp.float32).reshape(B, N, D), aux[0, 0]


# speedup vs baseline: 3.0448x; 3.0448x over previous
import jax
import jax.numpy as jnp
from jax import lax
from jax.experimental import pallas as pl
from jax.experimental.pallas import tpu as pltpu
from jax.experimental.pallas import tpu_sc as plsc

B, N, D = 4, 2048, 1024
F = 2048
E = 8
T = B * N
TM = 256
NET = 2 * T // TM + E
GROWS = NET * TM
TMR = 1024
TMS = 1024
NW = 32
TOKW = T // NW
NTM = 80
NEG = -1e30


def _router_body(x_ref, wr_ref, br_ref, e01_ref, w01_ref, rank_ref, cnt_ref,
                 aux_ref, acc_ref):
    i = pl.program_id(0)
    nb = pl.num_programs(0)
    x = x_ref[...]
    logits = jnp.dot(
        x.astype(jnp.bfloat16), wr_ref[...].astype(jnp.bfloat16),
        preferred_element_type=jnp.float32) + br_ref[...]
    mx = jnp.max(logits, axis=-1, keepdims=True)
    ex = jnp.exp(logits - mx)
    z = jnp.sum(ex, axis=-1, keepdims=True)
    probs = ex / z
    lane = jax.lax.broadcasted_iota(jnp.int32, logits.shape, 1)
    i0 = jnp.min(jnp.where(logits == mx, lane, E), axis=-1, keepdims=True)
    logits1 = jnp.where(lane == i0, NEG, logits)
    m1 = jnp.max(logits1, axis=-1, keepdims=True)
    i1 = jnp.min(jnp.where(logits1 == m1, lane, E), axis=-1, keepdims=True)
    p0 = 1.0 / z
    p1 = jnp.exp(m1 - mx) / z
    denom = p0 + p1 + 1e-9
    w0 = p0 / denom
    w1 = p1 / denom

    @pl.when(i == 0)
    def _():
        acc_ref[...] = jnp.zeros_like(acc_ref)

    oh0 = lane == i0
    oh1 = lane == i1
    oh01 = (oh0 | oh1).astype(jnp.bfloat16)
    r_iota = jax.lax.broadcasted_iota(jnp.int32, (TMR, TMR), 0)
    c_iota = jax.lax.broadcasted_iota(jnp.int32, (TMR, TMR), 1)
    ltri = (c_iota < r_iota).astype(jnp.bfloat16)
    cexcl = jnp.dot(ltri, oh01, preferred_element_type=jnp.float32)
    rankmat = cexcl + acc_ref[1:2, 0:E]
    rank0 = jnp.sum(jnp.where(oh0, rankmat, 0.0), axis=-1, keepdims=True)
    rank1 = jnp.sum(jnp.where(oh1, rankmat, 0.0), axis=-1, keepdims=True)

    lane2 = jax.lax.broadcasted_iota(jnp.int32, (TMR, 2), 1)
    e01_ref[...] = jnp.where(lane2 == 0, i0, i1)
    w01_ref[...] = jnp.where(lane2 == 0, w0, w1)
    rank_ref[...] = jnp.where(lane2 == 0, rank0, rank1).astype(jnp.int32)

    lane16 = jax.lax.broadcasted_iota(jnp.int32, (TMR, 16), 1)
    cnt16 = jnp.sum(
        (lane16 == i0).astype(jnp.float32) + (lane16 == i1).astype(jnp.float32),
        axis=0, keepdims=True)
    ps = jnp.sum(probs, axis=0, keepdims=True)
    acc_ref[0:1, 0:E] += ps
    acc_ref[1:2, 0:16] += cnt16

    @pl.when(i == nb - 1)
    def _():
        cnt_ref[...] = acc_ref[1:2, 0:16].astype(jnp.int32)
        imp = acc_ref[0:1, 0:E] / T
        load = acc_ref[1:2, 0:E] / (T * 2)
        aux_ref[...] = E * jnp.sum(imp * load, axis=-1, keepdims=True)


def _router(x_flat, Wr, br2):
    return pl.pallas_call(
        _router_body,
        grid=(T // TMR,),
        in_specs=[
            pl.BlockSpec((TMR, D), lambda i: (i, 0)),
            pl.BlockSpec((D, E), lambda i: (0, 0)),
            pl.BlockSpec((1, E), lambda i: (0, 0)),
        ],
        out_specs=[
            pl.BlockSpec((TMR, 2), lambda i: (i, 0)),
            pl.BlockSpec((TMR, 2), lambda i: (i, 0)),
            pl.BlockSpec((TMR, 2), lambda i: (i, 0)),
            pl.BlockSpec((1, 16), lambda i: (0, 0)),
            pl.BlockSpec((1, 1), lambda i: (0, 0)),
        ],
        out_shape=[
            jax.ShapeDtypeStruct((T, 2), jnp.int32),
            jax.ShapeDtypeStruct((T, 2), jnp.float32),
            jax.ShapeDtypeStruct((T, 2), jnp.int32),
            jax.ShapeDtypeStruct((1, 16), jnp.int32),
            jax.ShapeDtypeStruct((1, 1), jnp.float32),
        ],
        scratch_shapes=[pltpu.VMEM((8, 128), jnp.float32)],
    )(x_flat, Wr, br2)


def _dispatch_body(x_hbm, e01_hbm, rank_hbm, w01_hbm, cnt_hbm,
                   gx_hbm, sw_hbm, tmap_hbm, dstw_hbm,
                   cnt_v, pstart_v, e_v, rank_v, w_v, dst_v, xb0, xb1, xb2,
                   tmap_v, semr, sems, semw):
    wid = lax.axis_index("s") * 2 + lax.axis_index("c")
    tok0 = wid * TOKW
    pltpu.sync_copy(cnt_hbm, cnt_v)
    cnt = cnt_v[...]
    rup = jnp.bitwise_and(cnt + (TM - 1), -TM)
    lane0 = lax.broadcasted_iota(jnp.int32, (16,), 0)
    pstart = jnp.zeros((16,), jnp.int32)
    for e in range(E):
        pe = jnp.sum(jnp.where(lane0 < e, rup, 0))
        pstart = pstart + jnp.where(lane0 == e, pe, 0)
    pstart_v[...] = pstart
    mh = []
    for k in range(2):
        mh.append(pltpu.async_copy(e01_hbm.at[k, pl.ds(tok0, TOKW)],
                                   e_v.at[pl.ds(k * TOKW, TOKW)], semr))
        mh.append(pltpu.async_copy(rank_hbm.at[k, pl.ds(tok0, TOKW)],
                                   rank_v.at[pl.ds(k * TOKW, TOKW)], semr))
        mh.append(pltpu.async_copy(w01_hbm.at[k, pl.ds(tok0, TOKW)],
                                   w_v.at[pl.ds(k * TOKW, TOKW)], semr))
    for h in mh:
        h.wait()

    for r in range(16):
        for v in range(2):
            base = r * 32 + v * 16
            ev = e_v[pl.ds(base, 16)]
            rv = rank_v[pl.ds(base, 16)]
            ps = plsc.load_gather(pstart_v, [ev])
            dst_v[r, pl.ds(v * 16, 16)] = ps + rv
    pltpu.sync_copy(dst_v, dstw_hbm.at[wid])

    wh = [pltpu.async_copy(w_v.at[pl.ds(r * 32, 32)],
                           sw_hbm.at[dst_v.at[r]], semw) for r in range(16)]
    xb = (xb0, xb1, xb2)
    rh = [None] * 8
    sh = [None] * 8
    rh[0] = pltpu.async_copy(x_hbm.at[pl.ds(tok0, 32)], xb[0], semr)
    rh[1] = pltpu.async_copy(x_hbm.at[pl.ds(tok0 + 32, 32)], xb[1], semr)
    for c in range(8):
        rh[c].wait()
        sh[c] = (pltpu.async_copy(xb[c % 3], gx_hbm.at[dst_v.at[c]], sems),
                 pltpu.async_copy(xb[c % 3], gx_hbm.at[dst_v.at[8 + c]], sems))
        if c + 2 < 8:
            if c >= 1:
                sh[c - 1][0].wait()
                sh[c - 1][1].wait()
            t1 = tok0 + (c + 2) * 32
            rh[c + 2] = pltpu.async_copy(x_hbm.at[pl.ds(t1, 32)],
                                         xb[(c + 2) % 3], semr)
    for c in (6, 7):
        sh[c][0].wait()
        sh[c][1].wait()
    for h in wh:
        h.wait()

    @pl.when(wid == 0)
    def _():
        tiles = jnp.right_shift(rup, 8)
        lane = lax.broadcasted_iota(jnp.int32, (16,), 0)
        for v in range(NTM // 16):
            acc = jnp.zeros((16,), jnp.int32)
            iv = lane + v * 16
            for e in range(E):
                ce = jnp.sum(jnp.where(lane <= e, tiles, 0))
                acc = acc + (iv >= ce).astype(jnp.int32)
            tmap_v[pl.ds(v * 16, 16)] = acc
        pltpu.sync_copy(tmap_v, tmap_hbm)


def _dispatch(x_flat, e01, rank, w01, cnt16):
    mesh = plsc.VectorSubcoreMesh(core_axis_name="c", subcore_axis_name="s")
    f = pl.kernel(
        _dispatch_body,
        mesh=mesh,
        compiler_params=pltpu.CompilerParams(needs_layout_passes=False),
        out_type=[
            jax.ShapeDtypeStruct((GROWS, D), jnp.float32),
            jax.ShapeDtypeStruct((GROWS,), jnp.float32),
            jax.ShapeDtypeStruct((NTM,), jnp.int32),
            jax.ShapeDtypeStruct((NW, 16, 32), jnp.int32),
        ],
        scratch_types=[
            pltpu.VMEM((16,), jnp.int32),
            pltpu.VMEM((16,), jnp.int32),
            pltpu.VMEM((2 * TOKW,), jnp.int32),
            pltpu.VMEM((2 * TOKW,), jnp.int32),
            pltpu.VMEM((2 * TOKW,), jnp.float32),
            pltpu.VMEM((16, 32), jnp.int32),
            pltpu.VMEM((32, D), jnp.float32),
            pltpu.VMEM((32, D), jnp.float32),
            pltpu.VMEM((32, D), jnp.float32),
            pltpu.VMEM((NTM,), jnp.int32),
            pltpu.SemaphoreType.DMA,
            pltpu.SemaphoreType.DMA,
            pltpu.SemaphoreType.DMA,
        ],
    )
    return f(x_flat, e01, rank, w01, cnt16)


def _ffn_body(tmap_ref, gx_ref, sw_ref, w1_ref, b1_ref, w2_ref, b2_ref,
              out_ref):
    x = gx_ref[...].astype(jnp.bfloat16)
    h = jnp.maximum(
        jnp.dot(x, w1_ref[0], preferred_element_type=jnp.float32) + b1_ref[0],
        0.0)
    o = jnp.dot(h.astype(jnp.bfloat16), w2_ref[0],
                preferred_element_type=jnp.float32) + b2_ref[0]
    out_ref[...] = o * sw_ref[...]


def _ffn(tmap, gx, sw2, W1all, b1all, W2all, b2all):
    grid_spec = pltpu.PrefetchScalarGridSpec(
        num_scalar_prefetch=1,
        grid=(NET,),
        in_specs=[
            pl.BlockSpec((TM, D), lambda i, tm: (i, 0)),
            pl.BlockSpec((TM, 1), lambda i, tm: (i, 0)),
            pl.BlockSpec((1, D, F), lambda i, tm: (tm[i], 0, 0)),
            pl.BlockSpec((1, 1, F), lambda i, tm: (tm[i], 0, 0)),
            pl.BlockSpec((1, F, D), lambda i, tm: (tm[i], 0, 0)),
            pl.BlockSpec((1, 1, D), lambda i, tm: (tm[i], 0, 0)),
        ],
        out_specs=pl.BlockSpec((TM, D), lambda i, tm: (i, 0)),
    )
    return pl.pallas_call(
        _ffn_body,
        grid_spec=grid_spec,
        out_shape=jax.ShapeDtypeStruct((GROWS, D), jnp.float32),
    )(tmap, gx, sw2, W1all, b1all, W2all, b2all)


def _sffn_body(x_ref, ws1_ref, bs1_ref, ws2_ref, bs2_ref, out_ref):
    x = x_ref[...].astype(jnp.bfloat16)
    h = jnp.maximum(
        jnp.dot(x, ws1_ref[...], preferred_element_type=jnp.float32)
        + bs1_ref[...], 0.0)
    out_ref[...] = jnp.dot(
        h.astype(jnp.bfloat16), ws2_ref[...],
        preferred_element_type=jnp.float32) + bs2_ref[...]


def _sffn(x_flat, Ws1b, bs1r, Ws2b, bs2r):
    return pl.pallas_call(
        _sffn_body,
        grid=(T // TMS,),
        in_specs=[
            pl.BlockSpec((TMS, D), lambda i: (i, 0)),
            pl.BlockSpec((D, F), lambda i: (0, 0)),
            pl.BlockSpec((1, F), lambda i: (0, 0)),
            pl.BlockSpec((F, D), lambda i: (0, 0)),
            pl.BlockSpec((1, D), lambda i: (0, 0)),
        ],
        out_specs=pl.BlockSpec((TMS, D), lambda i: (i, 0)),
        out_shape=jax.ShapeDtypeStruct((T, D), jnp.float32),
    )(x_flat, Ws1b, bs1r, Ws2b, bs2r)


def _combine_body(fo_hbm, sh_hbm, dstw_hbm, y_hbm,
                  dst_v, b0a, b0b, b1a, b1b, bs_, ya, yb_,
                  semg, semy):
    wid = lax.axis_index("s") * 2 + lax.axis_index("c")
    tok0 = wid * TOKW
    pltpu.sync_copy(dstw_hbm.at[wid], dst_v)
    b0 = (b0a, b0b)
    b1 = (b1a, b1b)
    yb = (ya, yb_)

    def _issue(c):
        cc, hh = c // 2, c % 2
        p = c % 2
        return (
            pltpu.async_copy(fo_hbm.at[dst_v.at[cc, pl.ds(hh * 16, 16)]],
                             b0[p], semg),
            pltpu.async_copy(fo_hbm.at[dst_v.at[8 + cc, pl.ds(hh * 16, 16)]],
                             b1[p], semg),
        )

    gh = [None] * 16
    wrh = [None] * 16
    gh[0] = _issue(0)
    for c in range(16):
        for h in gh[c]:
            h.wait()
        if c + 1 < 16:
            gh[c + 1] = _issue(c + 1)
        p = c % 2
        pltpu.sync_copy(sh_hbm.at[pl.ds(tok0 + c * 16, 16)], bs_)
        if c >= 2:
            wrh[c - 2].wait()

        def _row(i, _, p=p):
            def _col(j, _, i=i):
                sl = pl.ds(j * 16, 16)
                yb[p][i, sl] = b0[p][i, sl] + b1[p][i, sl] + bs_[i, sl]
                return 0
            return lax.fori_loop(0, D // 16, _col, 0)

        lax.fori_loop(0, 16, _row, 0)
        wrh[c] = pltpu.async_copy(yb[p], y_hbm.at[pl.ds(tok0 + c * 16, 16)],
                                  semy)
    wrh[14].wait()
    wrh[15].wait()


def _combine(fo, sh, dstw):
    mesh = plsc.VectorSubcoreMesh(core_axis_name="c", subcore_axis_name="s")
    f = pl.kernel(
        _combine_body,
        mesh=mesh,
        compiler_params=pltpu.CompilerParams(needs_layout_passes=False),
        out_type=jax.ShapeDtypeStruct((T, D), jnp.float32),
        scratch_types=[
            pltpu.VMEM((16, 32), jnp.int32),
            pltpu.VMEM((16, D), jnp.float32),
            pltpu.VMEM((16, D), jnp.float32),
            pltpu.VMEM((16, D), jnp.float32),
            pltpu.VMEM((16, D), jnp.float32),
            pltpu.VMEM((16, D), jnp.float32),
            pltpu.VMEM((16, D), jnp.float32),
            pltpu.VMEM((16, D), jnp.float32),
            pltpu.SemaphoreType.DMA,
            pltpu.SemaphoreType.DMA,
        ],
    )
    return f(fo, sh, dstw)


def kernel(x, Wr, br, W1, b1, W2, b2, Ws1, bs1, Ws2, bs2):
    x_flat = x.reshape(T, D)
    W1all = jnp.concatenate([W1, Ws1[None]], axis=0).astype(jnp.bfloat16)
    W2all = jnp.concatenate([W2, Ws2[None]], axis=0).astype(jnp.bfloat16)
    b1all = jnp.concatenate([b1, bs1[None]], axis=0).reshape(E + 1, 1, F)
    b2all = jnp.concatenate([b2, bs2[None]], axis=0).reshape(E + 1, 1, D)
    br2 = br.reshape(1, E)

    e01t, w01t, rankt, cnt16, aux = _router(x_flat, Wr, br2)
    e01 = e01t.T
    w01 = w01t.T
    rank = rankt.T
    sh = _sffn(x_flat, Ws1.astype(jnp.bfloat16), bs1.reshape(1, F),
               Ws2.astype(jnp.bfloat16), bs2.reshape(1, D))
    gx, sw, tmap, dstw = _dispatch(x_flat, e01, rank, w01, cnt16.reshape(16))
    fo = _ffn(tmap, gx, sw.reshape(GROWS, 1), W1all, b1all, W2all, b2all)
    y = _combine(fo, sh, dstw)
    return y.reshape(B, N, D), aux[0, 0]


# speedup vs baseline: 3.0832x; 1.0126x over previous
import jax
import jax.numpy as jnp
from jax import lax
from jax.experimental import pallas as pl
from jax.experimental.pallas import tpu as pltpu
from jax.experimental.pallas import tpu_sc as plsc

B, N, D = 4, 2048, 1024
F = 2048
E = 8
T = B * N
TM = 256
NET = 2 * T // TM + E
GROWS = NET * TM
TMR = 1024
TMS = 1024
NW = 32
TOKW = T // NW
NTM = 80
NEG = -1e30


def _router_body(x_ref, wr_ref, br_ref, e01_ref, w01_ref, rank_ref, cnt_ref,
                 aux_ref, acc_ref):
    i = pl.program_id(0)
    nb = pl.num_programs(0)
    x = x_ref[...]
    logits = jnp.dot(
        x.astype(jnp.bfloat16), wr_ref[...].astype(jnp.bfloat16),
        preferred_element_type=jnp.float32) + br_ref[...]
    mx = jnp.max(logits, axis=-1, keepdims=True)
    ex = jnp.exp(logits - mx)
    z = jnp.sum(ex, axis=-1, keepdims=True)
    probs = ex / z
    lane = jax.lax.broadcasted_iota(jnp.int32, logits.shape, 1)
    i0 = jnp.min(jnp.where(logits == mx, lane, E), axis=-1, keepdims=True)
    logits1 = jnp.where(lane == i0, NEG, logits)
    m1 = jnp.max(logits1, axis=-1, keepdims=True)
    i1 = jnp.min(jnp.where(logits1 == m1, lane, E), axis=-1, keepdims=True)
    p0 = 1.0 / z
    p1 = jnp.exp(m1 - mx) / z
    denom = p0 + p1 + 1e-9
    w0 = p0 / denom
    w1 = p1 / denom

    @pl.when(i == 0)
    def _():
        acc_ref[...] = jnp.zeros_like(acc_ref)

    oh0 = lane == i0
    oh1 = lane == i1
    oh01 = (oh0 | oh1).astype(jnp.bfloat16)
    r_iota = jax.lax.broadcasted_iota(jnp.int32, (TMR, TMR), 0)
    c_iota = jax.lax.broadcasted_iota(jnp.int32, (TMR, TMR), 1)
    ltri = (c_iota < r_iota).astype(jnp.bfloat16)
    cexcl = jnp.dot(ltri, oh01, preferred_element_type=jnp.float32)
    rankmat = cexcl + acc_ref[1:2, 0:E]
    rank0 = jnp.sum(jnp.where(oh0, rankmat, 0.0), axis=-1, keepdims=True)
    rank1 = jnp.sum(jnp.where(oh1, rankmat, 0.0), axis=-1, keepdims=True)

    lane2 = jax.lax.broadcasted_iota(jnp.int32, (TMR, 2), 1)
    e01_ref[...] = jnp.where(lane2 == 0, i0, i1)
    w01_ref[...] = jnp.where(lane2 == 0, w0, w1)
    rank_ref[...] = jnp.where(lane2 == 0, rank0, rank1).astype(jnp.int32)

    lane16 = jax.lax.broadcasted_iota(jnp.int32, (TMR, 16), 1)
    cnt16 = jnp.sum(
        (lane16 == i0).astype(jnp.float32) + (lane16 == i1).astype(jnp.float32),
        axis=0, keepdims=True)
    ps = jnp.sum(probs, axis=0, keepdims=True)
    acc_ref[0:1, 0:E] += ps
    acc_ref[1:2, 0:16] += cnt16

    @pl.when(i == nb - 1)
    def _():
        cnt_ref[...] = acc_ref[1:2, 0:16].astype(jnp.int32)
        imp = acc_ref[0:1, 0:E] / T
        load = acc_ref[1:2, 0:E] / (T * 2)
        aux_ref[...] = E * jnp.sum(imp * load, axis=-1, keepdims=True)


def _router(x_flat, Wr, br2):
    return pl.pallas_call(
        _router_body,
        grid=(T // TMR,),
        in_specs=[
            pl.BlockSpec((TMR, D), lambda i: (i, 0)),
            pl.BlockSpec((D, E), lambda i: (0, 0)),
            pl.BlockSpec((1, E), lambda i: (0, 0)),
        ],
        out_specs=[
            pl.BlockSpec((TMR, 2), lambda i: (i, 0)),
            pl.BlockSpec((TMR, 2), lambda i: (i, 0)),
            pl.BlockSpec((TMR, 2), lambda i: (i, 0)),
            pl.BlockSpec((1, 16), lambda i: (0, 0)),
            pl.BlockSpec((1, 1), lambda i: (0, 0)),
        ],
        out_shape=[
            jax.ShapeDtypeStruct((T, 2), jnp.int32),
            jax.ShapeDtypeStruct((T, 2), jnp.float32),
            jax.ShapeDtypeStruct((T, 2), jnp.int32),
            jax.ShapeDtypeStruct((1, 16), jnp.int32),
            jax.ShapeDtypeStruct((1, 1), jnp.float32),
        ],
        scratch_shapes=[pltpu.VMEM((8, 128), jnp.float32)],
    )(x_flat, Wr, br2)


def _dispatch_body(x_hbm, e01_hbm, rank_hbm, w01_hbm, cnt_hbm,
                   gx_hbm, sw_hbm, tmap_hbm, dstw_hbm,
                   cnt_v, pstart_v, e_v, rank_v, w_v, dst_v, xb0, xb1,
                   tmap_v, semr, sems, semw):
    wid = lax.axis_index("s") * 2 + lax.axis_index("c")
    tok0 = wid * TOKW
    pltpu.sync_copy(cnt_hbm, cnt_v)
    cnt = cnt_v[...]
    rup = jnp.bitwise_and(cnt + (TM - 1), -TM)
    lane0 = lax.broadcasted_iota(jnp.int32, (16,), 0)
    pstart = jnp.zeros((16,), jnp.int32)
    for e in range(E):
        pe = jnp.sum(jnp.where(lane0 < e, rup, 0))
        pstart = pstart + jnp.where(lane0 == e, pe, 0)
    pstart_v[...] = pstart
    for k in range(2):
        pltpu.sync_copy(e01_hbm.at[k, pl.ds(tok0, TOKW)],
                        e_v.at[pl.ds(k * TOKW, TOKW)])
        pltpu.sync_copy(rank_hbm.at[k, pl.ds(tok0, TOKW)],
                        rank_v.at[pl.ds(k * TOKW, TOKW)])
        pltpu.sync_copy(w01_hbm.at[k, pl.ds(tok0, TOKW)],
                        w_v.at[pl.ds(k * TOKW, TOKW)])

    for r in range(16):
        for v in range(2):
            base = r * 32 + v * 16
            ev = e_v[pl.ds(base, 16)]
            rv = rank_v[pl.ds(base, 16)]
            ps = plsc.load_gather(pstart_v, [ev])
            dst_v[r, pl.ds(v * 16, 16)] = ps + rv
    pltpu.sync_copy(dst_v, dstw_hbm.at[wid])

    wh = [pltpu.async_copy(w_v.at[pl.ds(r * 32, 32)],
                           sw_hbm.at[dst_v.at[r]], semw) for r in range(16)]
    xb = (xb0, xb1)
    rh = [None] * 8
    sh = [None] * 8
    rh[0] = pltpu.async_copy(x_hbm.at[pl.ds(tok0, 32)], xb[0], semr)
    for c in range(8):
        rh[c].wait()
        sh[c] = (pltpu.async_copy(xb[c % 2], gx_hbm.at[dst_v.at[c]], sems),
                 pltpu.async_copy(xb[c % 2], gx_hbm.at[dst_v.at[8 + c]], sems))
        if c + 1 < 8:
            if c >= 1:
                sh[c - 1][0].wait()
                sh[c - 1][1].wait()
            t1 = tok0 + (c + 1) * 32
            rh[c + 1] = pltpu.async_copy(x_hbm.at[pl.ds(t1, 32)],
                                         xb[(c + 1) % 2], semr)
    sh[6][0].wait()
    sh[6][1].wait()
    sh[7][0].wait()
    sh[7][1].wait()
    for h in wh:
        h.wait()

    @pl.when(wid == 0)
    def _():
        tiles = jnp.right_shift(rup, 8)
        lane = lax.broadcasted_iota(jnp.int32, (16,), 0)
        for v in range(NTM // 16):
            acc = jnp.zeros((16,), jnp.int32)
            iv = lane + v * 16
            for e in range(E):
                ce = jnp.sum(jnp.where(lane <= e, tiles, 0))
                acc = acc + (iv >= ce).astype(jnp.int32)
            tmap_v[pl.ds(v * 16, 16)] = acc
        pltpu.sync_copy(tmap_v, tmap_hbm)


def _dispatch(x_flat, e01, rank, w01, cnt16):
    mesh = plsc.VectorSubcoreMesh(core_axis_name="c", subcore_axis_name="s")
    f = pl.kernel(
        _dispatch_body,
        mesh=mesh,
        compiler_params=pltpu.CompilerParams(needs_layout_passes=False),
        out_type=[
            jax.ShapeDtypeStruct((GROWS, D), jnp.float32),
            jax.ShapeDtypeStruct((GROWS,), jnp.float32),
            jax.ShapeDtypeStruct((NTM,), jnp.int32),
            jax.ShapeDtypeStruct((NW, 16, 32), jnp.int32),
        ],
        scratch_types=[
            pltpu.VMEM((16,), jnp.int32),
            pltpu.VMEM((16,), jnp.int32),
            pltpu.VMEM((2 * TOKW,), jnp.int32),
            pltpu.VMEM((2 * TOKW,), jnp.int32),
            pltpu.VMEM((2 * TOKW,), jnp.float32),
            pltpu.VMEM((16, 32), jnp.int32),
            pltpu.VMEM((32, D), jnp.float32),
            pltpu.VMEM((32, D), jnp.float32),
            pltpu.VMEM((NTM,), jnp.int32),
            pltpu.SemaphoreType.DMA,
            pltpu.SemaphoreType.DMA,
            pltpu.SemaphoreType.DMA,
        ],
    )
    return f(x_flat, e01, rank, w01, cnt16)


def _ffn_body(tmap_ref, gx_ref, sw_ref, w1_ref, b1_ref, w2_ref, b2_ref,
              out_ref):
    x = gx_ref[...].astype(jnp.bfloat16)
    h = jnp.maximum(
        jnp.dot(x, w1_ref[0], preferred_element_type=jnp.float32) + b1_ref[0],
        0.0)
    o = jnp.dot(h.astype(jnp.bfloat16), w2_ref[0],
                preferred_element_type=jnp.float32) + b2_ref[0]
    out_ref[...] = o * sw_ref[...]


def _ffn(tmap, gx, sw2, W1all, b1all, W2all, b2all):
    grid_spec = pltpu.PrefetchScalarGridSpec(
        num_scalar_prefetch=1,
        grid=(NET,),
        in_specs=[
            pl.BlockSpec((TM, D), lambda i, tm: (i, 0)),
            pl.BlockSpec((TM, 1), lambda i, tm: (i, 0)),
            pl.BlockSpec((1, D, F), lambda i, tm: (tm[i], 0, 0)),
            pl.BlockSpec((1, 1, F), lambda i, tm: (tm[i], 0, 0)),
            pl.BlockSpec((1, F, D), lambda i, tm: (tm[i], 0, 0)),
            pl.BlockSpec((1, 1, D), lambda i, tm: (tm[i], 0, 0)),
        ],
        out_specs=pl.BlockSpec((TM, D), lambda i, tm: (i, 0)),
    )
    return pl.pallas_call(
        _ffn_body,
        grid_spec=grid_spec,
        out_shape=jax.ShapeDtypeStruct((GROWS, D), jnp.float32),
    )(tmap, gx, sw2, W1all, b1all, W2all, b2all)


def _sffn_body(x_ref, ws1_ref, bs1_ref, ws2_ref, bs2_ref, out_ref):
    x = x_ref[...].astype(jnp.bfloat16)
    h = jnp.maximum(
        jnp.dot(x, ws1_ref[...], preferred_element_type=jnp.float32)
        + bs1_ref[...], 0.0)
    out_ref[...] = jnp.dot(
        h.astype(jnp.bfloat16), ws2_ref[...],
        preferred_element_type=jnp.float32) + bs2_ref[...]


def _sffn(x_flat, Ws1b, bs1r, Ws2b, bs2r):
    return pl.pallas_call(
        _sffn_body,
        grid=(T // TMS,),
        in_specs=[
            pl.BlockSpec((TMS, D), lambda i: (i, 0)),
            pl.BlockSpec((D, F), lambda i: (0, 0)),
            pl.BlockSpec((1, F), lambda i: (0, 0)),
            pl.BlockSpec((F, D), lambda i: (0, 0)),
            pl.BlockSpec((1, D), lambda i: (0, 0)),
        ],
        out_specs=pl.BlockSpec((TMS, D), lambda i: (i, 0)),
        out_shape=jax.ShapeDtypeStruct((T, D), jnp.float32),
    )(x_flat, Ws1b, bs1r, Ws2b, bs2r)


def _combine_body(fo_hbm, sh_hbm, dstw_hbm, y_hbm,
                  dst_v, b0a, b0b, b1a, b1b, bs_, ya, yb_,
                  semg, semy):
    wid = lax.axis_index("s") * 2 + lax.axis_index("c")
    tok0 = wid * TOKW
    pltpu.sync_copy(dstw_hbm.at[wid], dst_v)
    b0 = (b0a, b0b)
    b1 = (b1a, b1b)
    yb = (ya, yb_)

    def _issue(c):
        cc, hh = c // 2, c % 2
        p = c % 2
        return (
            pltpu.async_copy(fo_hbm.at[dst_v.at[cc, pl.ds(hh * 16, 16)]],
                             b0[p], semg),
            pltpu.async_copy(fo_hbm.at[dst_v.at[8 + cc, pl.ds(hh * 16, 16)]],
                             b1[p], semg),
        )

    gh = [None] * 16
    wrh = [None] * 16
    gh[0] = _issue(0)
    for c in range(16):
        for h in gh[c]:
            h.wait()
        if c + 1 < 16:
            gh[c + 1] = _issue(c + 1)
        p = c % 2
        pltpu.sync_copy(sh_hbm.at[pl.ds(tok0 + c * 16, 16)], bs_)
        if c >= 2:
            wrh[c - 2].wait()

        def _row(i, _, p=p):
            def _col(j, _, i=i):
                sl = pl.ds(j * 16, 16)
                yb[p][i, sl] = b0[p][i, sl] + b1[p][i, sl] + bs_[i, sl]
                return 0
            return lax.fori_loop(0, D // 16, _col, 0)

        lax.fori_loop(0, 16, _row, 0)
        wrh[c] = pltpu.async_copy(yb[p], y_hbm.at[pl.ds(tok0 + c * 16, 16)],
                                  semy)
    wrh[14].wait()
    wrh[15].wait()


def _combine(fo, sh, dstw):
    mesh = plsc.VectorSubcoreMesh(core_axis_name="c", subcore_axis_name="s")
    f = pl.kernel(
        _combine_body,
        mesh=mesh,
        compiler_params=pltpu.CompilerParams(needs_layout_passes=False),
        out_type=jax.ShapeDtypeStruct((T, D), jnp.float32),
        scratch_types=[
            pltpu.VMEM((16, 32), jnp.int32),
            pltpu.VMEM((16, D), jnp.float32),
            pltpu.VMEM((16, D), jnp.float32),
            pltpu.VMEM((16, D), jnp.float32),
            pltpu.VMEM((16, D), jnp.float32),
            pltpu.VMEM((16, D), jnp.float32),
            pltpu.VMEM((16, D), jnp.float32),
            pltpu.VMEM((16, D), jnp.float32),
            pltpu.SemaphoreType.DMA,
            pltpu.SemaphoreType.DMA,
        ],
    )
    return f(fo, sh, dstw)


def kernel(x, Wr, br, W1, b1, W2, b2, Ws1, bs1, Ws2, bs2):
    x_flat = x.reshape(T, D)
    W1all = jnp.concatenate([W1, Ws1[None]], axis=0).astype(jnp.bfloat16)
    W2all = jnp.concatenate([W2, Ws2[None]], axis=0).astype(jnp.bfloat16)
    b1all = jnp.concatenate([b1, bs1[None]], axis=0).reshape(E + 1, 1, F)
    b2all = jnp.concatenate([b2, bs2[None]], axis=0).reshape(E + 1, 1, D)
    br2 = br.reshape(1, E)

    e01t, w01t, rankt, cnt16, aux = _router(x_flat, Wr, br2)
    e01 = e01t.T
    w01 = w01t.T
    rank = rankt.T
    gx, sw, tmap, dstw = _dispatch(x_flat, e01, rank, w01, cnt16.reshape(16))
    sh = _sffn(x_flat, Ws1.astype(jnp.bfloat16), bs1.reshape(1, F),
               Ws2.astype(jnp.bfloat16), bs2.reshape(1, D))
    fo = _ffn(tmap, gx, sw.reshape(GROWS, 1), W1all, b1all, W2all, b2all)
    y = _combine(fo, sh, dstw)
    return y.reshape(B, N, D), aux[0, 0]
